# baseline (speedup 1.0000x reference)
import functools

import jax
import jax.numpy as jnp
from jax import lax
from jax.experimental import pallas as pl
from jax.experimental.pallas import tpu as pltpu
from jax.experimental.pallas import tpu_sc as plsc

_N = 10000
_E = 320000
_D = 128
_DE = 16


def _ln_relu(h):
    m = jnp.mean(h, axis=-1, keepdims=True)
    v = jnp.mean((h - m) * (h - m), axis=-1, keepdims=True)
    return jnp.maximum((h - m) / jnp.sqrt(v + 1e-5), 0.0)


def _ln_relu_mxu(h, ones):
    s = jnp.dot(h, ones, preferred_element_type=jnp.float32)
    q = jnp.dot(h * h, ones, preferred_element_type=jnp.float32)
    m = s * (1.0 / _D)
    v = q * (1.0 / _D) - m * m
    return jnp.maximum((h - m) * lax.rsqrt(v + 1e-5), 0.0)


def _dot(a, b):
    return jnp.dot(a, b, preferred_element_type=jnp.float32)


_BN_PREP = 2000


def _prep_body(x_ref, ws_ref, wd_ref, wn_ref, g_ref, wge_ref, wgn_ref,
               be_ref, bn_ref, xs_ref, xd_ref, xn_ref, ge_ref, gn_ref):
    xb = x_ref[...]
    xs_ref[...] = _dot(xb, ws_ref[...])
    xd_ref[...] = _dot(xb, wd_ref[...])
    xn_ref[...] = _dot(xb, wn_ref[...])

    @pl.when(pl.program_id(0) == 0)
    def _():
        g = g_ref[...]
        ge_ref[...] = _dot(g, wge_ref[...]) + be_ref[...]
        gn_ref[...] = _dot(g, wgn_ref[...]) + bn_ref[...]


def _prep(x, w1s, w1d, wnx, g, w1g, wng, be1, bn1):
    full = lambda shp: pl.BlockSpec(shp, lambda i: (0, 0))
    return pl.pallas_call(
        _prep_body,
        grid=(_N // _BN_PREP,),
        in_specs=[
            pl.BlockSpec((_BN_PREP, _D), lambda i: (i, 0)),
            full((_D, _D)), full((_D, _D)), full((_D, _D)),
            full((1, _D)), full((_D, _D)), full((_D, _D)),
            full((1, _D)), full((1, _D)),
        ],
        out_specs=[
            pl.BlockSpec((_BN_PREP, _D), lambda i: (i, 0)),
            pl.BlockSpec((_BN_PREP, _D), lambda i: (i, 0)),
            pl.BlockSpec((_BN_PREP, _D), lambda i: (i, 0)),
            full((1, _D)), full((1, _D)),
        ],
        out_shape=[
            jax.ShapeDtypeStruct((_N, _D), jnp.float32),
            jax.ShapeDtypeStruct((_N, _D), jnp.float32),
            jax.ShapeDtypeStruct((_N, _D), jnp.float32),
            jax.ShapeDtypeStruct((1, _D), jnp.float32),
            jax.ShapeDtypeStruct((1, _D), jnp.float32),
        ],
    )(x, w1s, w1d, wnx, g, w1g, wng, be1, bn1)


_NW = 32
_EPW = _E // _NW
_GC = 80
_GNC = _EPW // _GC


def _sc_gather(xs, xd, src, dst, ebase, size):
    epw = size // _NW
    gnc = epw // _GC
    mesh = plsc.VectorSubcoreMesh(core_axis_name="c", subcore_axis_name="s")

    @functools.partial(
        pl.kernel,
        out_type=jax.ShapeDtypeStruct((size, _D), jnp.float32),
        mesh=mesh,
        scratch_types=[
            pltpu.VMEM((epw,), jnp.int32),
            pltpu.VMEM((epw,), jnp.int32),
            pltpu.VMEM((_GC, _D), jnp.float32),
            pltpu.VMEM((_GC, _D), jnp.float32),
            pltpu.VMEM((_GC, _D), jnp.float32),
            pltpu.VMEM((_GC, _D), jnp.float32),
            pltpu.SemaphoreType.DMA,
            pltpu.SemaphoreType.DMA,
            pltpu.SemaphoreType.DMA,
            pltpu.SemaphoreType.DMA,
            pltpu.SemaphoreType.DMA,
            pltpu.SemaphoreType.DMA,
        ],
    )
    def k(xs_hbm, xd_hbm, src_hbm, dst_hbm, out_hbm,
          ias, iad, rs0, rs1, rd0, rd1, sgs0, sgs1, sgd0, sgd1, so0, so1):
        c = lax.axis_index("c")
        s = lax.axis_index("s")
        base = (s * 2 + c) * epw
        rs, rd = (rs0, rs1), (rd0, rd1)
        sgs, sgd, so = (sgs0, sgs1), (sgd0, sgd1), (so0, so1)

        pltpu.sync_copy(src_hbm.at[pl.ds(ebase + base, epw)], ias)
        pltpu.sync_copy(dst_hbm.at[pl.ds(ebase + base, epw)], iad)

        def issue_gather(j, p):
            pltpu.async_copy(xs_hbm.at[ias.at[pl.ds(j * _GC, _GC)]], rs[p], sgs[p])
            pltpu.async_copy(xd_hbm.at[iad.at[pl.ds(j * _GC, _GC)]], rd[p], sgd[p])

        issue_gather(0, 0)

        def body(m, carry):
            for p in (0, 1):
                j = 2 * m + p
                p1 = 1 - p

                @pl.when(j <= gnc - 1)
                def _():
                    pltpu.make_async_copy(
                        xs_hbm.at[ias.at[pl.ds(0, _GC)]], rs[p], sgs[p]).wait()
                    pltpu.make_async_copy(
                        xd_hbm.at[iad.at[pl.ds(0, _GC)]], rd[p], sgd[p]).wait()

                    @pl.when(j >= 1)
                    def _():
                        pltpu.make_async_copy(
                            rs[p1], out_hbm.at[pl.ds(base, _GC)], so[p1]).wait()

                    @pl.when(j + 1 <= gnc - 1)
                    def _():
                        issue_gather(j + 1, p1)

                    def addrow(i, c2):
                        for kk in range(_D // 16):
                            sl = pl.ds(kk * 16, 16)
                            rs[p][i, sl] = rs[p][i, sl] + rd[p][i, sl]
                        return c2

                    lax.fori_loop(0, _GC, addrow, 0)
                    pltpu.async_copy(
                        rs[p], out_hbm.at[pl.ds(base + j * _GC, _GC)], so[p])
            return carry

        lax.fori_loop(0, (gnc + 1) // 2, body, 0)
        pltpu.make_async_copy(
            rs[(gnc - 1) % 2], out_hbm.at[pl.ds(base, _GC)],
            so[(gnc - 1) % 2]).wait()

    return k(xs, xd, src, dst)


_AGG_NC = _N // _GC


def _sc_scatter(eout, dst, ebase, size):
    epw = size // _NW
    gnc = epw // _GC
    mesh = plsc.VectorSubcoreMesh(core_axis_name="c", subcore_axis_name="s")

    @functools.partial(
        pl.kernel,
        out_type=jax.ShapeDtypeStruct((2, _N, _D), jnp.float32),
        mesh=mesh,
        scratch_types=[
            pltpu.VMEM((_GC,), jnp.int32),
            pltpu.VMEM((_GC,), jnp.int32),
            pltpu.VMEM((_GC, _D), jnp.float32),
            pltpu.VMEM((_GC, _D), jnp.float32),
            pltpu.VMEM_SHARED((_N, _D), jnp.float32),
            pltpu.SemaphoreType.DMA,
            pltpu.SemaphoreType.DMA,
            pltpu.SemaphoreType.DMA,
            pltpu.SemaphoreType.DMA,
        ],
    )
    def k(eout_hbm, dst_hbm, out_hbm, i0, i1, r0, r1, agg_sh, sr0, sr1, si0, si1):
        c = lax.axis_index("c")
        s = lax.axis_index("s")
        ib, rb = (i0, i1), (r0, r1)
        sr, si = (sr0, sr1), (si0, si1)
        nq = jnp.where(s <= (_AGG_NC - 1) % 16, _AGG_NC // 16 + 1, _AGG_NC // 16)

        def zrow(i, carry):
            for kk in range(_D // 16):
                r0[i, pl.ds(kk * 16, 16)] = jnp.zeros((16,), jnp.float32)
            return carry

        lax.fori_loop(0, _GC, zrow, 0)

        def zchunk(m, carry):
            q = s + 16 * m
            pltpu.sync_copy(r0, agg_sh.at[pl.ds(q * _GC, _GC)])
            return carry

        lax.fori_loop(0, nq, zchunk, 0)
        plsc.subcore_barrier()

        w = c * 16 + s
        base = w * epw
        pltpu.async_copy(dst_hbm.at[pl.ds(ebase + base, _GC)], i0, si0)
        pltpu.async_copy(eout_hbm.at[pl.ds(base, _GC)], r0, sr0)

        def body(m, carry):
            for p in (0, 1):
                j = 2 * m + p
                p1 = 1 - p

                @pl.when(j <= gnc - 1)
                def _():
                    pltpu.make_async_copy(
                        dst_hbm.at[pl.ds(base, _GC)], ib[p], si[p]).wait()
                    pltpu.make_async_copy(
                        eout_hbm.at[pl.ds(base, _GC)], rb[p], sr[p]).wait()

                    @pl.when(j + 1 <= gnc - 1)
                    def _():
                        off1 = base + (j + 1) * _GC
                        pltpu.async_copy(dst_hbm.at[pl.ds(ebase + off1, _GC)],
                                         ib[p1], si[p1])
                        pltpu.async_copy(eout_hbm.at[pl.ds(off1, _GC)],
                                         rb[p1], sr[p1])

                    pltpu.sync_copy(rb[p], agg_sh.at[ib[p]], add=True)
            return carry

        lax.fori_loop(0, (gnc + 1) // 2, body, 0)
        plsc.subcore_barrier()

        def wb(m, carry):
            q = s + 16 * m
            pltpu.sync_copy(agg_sh.at[pl.ds(q * _GC, _GC)],
                            out_hbm.at[c, pl.ds(q * _GC, _GC)])
            return carry

        lax.fori_loop(0, nq, wb, 0)

    return k(eout, dst)


_BE = 6400
_S0 = 26 * _BE
_S1 = _E - _S0


def _edge_body(ea_ref, xsum_ref, we_ref, w2_ref, w3_ref, w4_ref,
               b2_ref, b3_ref, b4_ref, ge_ref, ones_ref, eout_ref, esum_ref,
               epriv_ref):
    onesb = ones_ref[...]

    def lnr(h):
        hb = h.astype(jnp.bfloat16)
        s = jnp.dot(hb, onesb, preferred_element_type=jnp.float32)
        q = jnp.dot(hb * hb, onesb, preferred_element_type=jnp.float32)
        m = s * (1.0 / _D)
        v = q * (1.0 / _D) - m * m
        return jnp.maximum((h - m) * lax.rsqrt(v + 1e-5), 0.0)

    ea_part = lax.dot_general(ea_ref[...].astype(jnp.bfloat16), we_ref[...],
                              (((0,), (0,)), ((), ())),
                              preferred_element_type=jnp.float32)
    h = ea_part + xsum_ref[...] + ge_ref[...]
    h = lnr(h)
    h = lnr(_dot(h.astype(jnp.bfloat16), w2_ref[...]) + b2_ref[...])
    h = lnr(_dot(h.astype(jnp.bfloat16), w3_ref[...]) + b3_ref[...])
    out = _dot(h.astype(jnp.bfloat16), w4_ref[...]) + b4_ref[...]
    eout_ref[...] = out
    epriv_ref[...] = out

    @pl.when(pl.program_id(0) == 0)
    def _():
        esum_ref[...] = jnp.zeros_like(esum_ref)

    esum_ref[...] += jnp.sum(out, axis=0, keepdims=True)


def _edge_body_alias(eprev_ref, *refs):
    _edge_body(*refs)


def _sum6(a0, a1, a2):
    return (a0[0] + a0[1]) + (a1[0] + a1[1]) + (a2[0] + a2[1])


def _edge_mlp(ea_t, xsum_h, w1e, w2, w3, w4, b2, b3, b4, ge, ones,
              blk_off, eprev=None):
    full = lambda shp: pl.BlockSpec(shp, lambda i: (0, 0))
    nblk = xsum_h.shape[0] // _BE
    in_specs = [
        pl.BlockSpec((_DE, _BE), lambda i, o=blk_off: (0, i + o)),
        pl.BlockSpec((_BE, _D), lambda i: (i, 0)),
        full((_DE, _D)), full((_D, _D)), full((_D, _D)), full((_D, _D)),
        full((1, _D)), full((1, _D)), full((1, _D)), full((1, _D)),
        full((_D, _D)),
    ]
    args = (ea_t, xsum_h, w1e, w2, w3, w4, b2, b3, b4, ge, ones)
    body = _edge_body
    aliases = {}
    if eprev is not None:
        in_specs = [pl.BlockSpec(memory_space=pl.ANY)] + in_specs
        args = (eprev,) + args
        body = _edge_body_alias
        aliases = {0: 0}
    return pl.pallas_call(
        body,
        grid=(nblk,),
        in_specs=in_specs,
        out_specs=[
            pl.BlockSpec((_BE, _D), lambda i, o=blk_off: (i + o, 0)),
            full((1, _D)),
            pl.BlockSpec((_BE, _D), lambda i: (i, 0)),
        ],
        out_shape=[
            jax.ShapeDtypeStruct((_E, _D), jnp.float32),
            jax.ShapeDtypeStruct((1, _D), jnp.float32),
            jax.ShapeDtypeStruct((nblk * _BE, _D), jnp.float32),
        ],
        input_output_aliases=aliases,
    )(*args)


_BN = 2000


def _node_body(xn_ref, aggp0_ref, aggp1_ref, aggp2_ref, wa_ref, w2_ref,
               w3_ref, w4_ref, b2_ref, b3_ref, b4_ref, gn_ref, ones_ref,
               nout_ref, nsum_ref):
    ones = ones_ref[...]
    agg = _sum6(aggp0_ref, aggp1_ref, aggp2_ref)
    h = xn_ref[...] + _dot(agg, wa_ref[...]) + gn_ref[...]
    h = _ln_relu_mxu(h, ones)
    h = _ln_relu_mxu(_dot(h, w2_ref[...]) + b2_ref[...], ones)
    h = _ln_relu_mxu(_dot(h, w3_ref[...]) + b3_ref[...], ones)
    out = _dot(h, w4_ref[...]) + b4_ref[...]
    nout_ref[...] = out

    @pl.when(pl.program_id(0) == 0)
    def _():
        nsum_ref[...] = jnp.zeros_like(nsum_ref)

    nsum_ref[...] += jnp.sum(out, axis=0, keepdims=True)


def _node_mlp(xn, aggps, wna, w2, w3, w4, b2, b3, b4, gn, ones):
    full = lambda shp: pl.BlockSpec(shp, lambda i: (0, 0))
    return pl.pallas_call(
        _node_body,
        grid=(_N // _BN,),
        in_specs=[
            pl.BlockSpec((_BN, _D), lambda i: (i, 0)),
            pl.BlockSpec((2, _BN, _D), lambda i: (0, i, 0)),
            pl.BlockSpec((2, _BN, _D), lambda i: (0, i, 0)),
            pl.BlockSpec((2, _BN, _D), lambda i: (0, i, 0)),
            full((_D, _D)), full((_D, _D)), full((_D, _D)), full((_D, _D)),
            full((1, _D)), full((1, _D)), full((1, _D)), full((1, _D)),
            full((_D, _D)),
        ],
        out_specs=[
            pl.BlockSpec((_BN, _D), lambda i: (i, 0)),
            full((1, _D)),
        ],
        out_shape=[
            jax.ShapeDtypeStruct((_N, _D), jnp.float32),
            jax.ShapeDtypeStruct((1, _D), jnp.float32),
        ],
    )(xn, *aggps, wna, w2, w3, w4, b2, b3, b4, gn, ones)


def _glob_body(nsum_ref, esum0_ref, esum1_ref, esum2_ref, g_ref, wgn_ref,
               wge_ref, wgg_ref, w2_ref, w3_ref, w4_ref, b1_ref, b2_ref,
               b3_ref, b4_ref, gout_ref):
    mn = nsum_ref[...] * (1.0 / _N)
    me = (esum0_ref[...] + esum1_ref[...] + esum2_ref[...]) * (1.0 / _E)
    h = (_dot(mn, wgn_ref[...]) + _dot(me, wge_ref[...])
         + _dot(g_ref[...], wgg_ref[...]) + b1_ref[...])
    h = _ln_relu(h)
    h = _ln_relu(_dot(h, w2_ref[...]) + b2_ref[...])
    h = _ln_relu(_dot(h, w3_ref[...]) + b3_ref[...])
    gout_ref[...] = _dot(h, w4_ref[...]) + b4_ref[...]


def _glob_mlp(nsum, esums, g, wgn, wge, wgg, w2, w3, w4, b1, b2, b3, b4):
    return pl.pallas_call(
        _glob_body,
        out_shape=jax.ShapeDtypeStruct((1, _D), jnp.float32),
    )(nsum, *esums, g, wgn, wge, wgg, w2, w3, w4, b1, b2, b3, b4)


def kernel(x, edge_index, edge_attr, graph_globals,
           edge_Ws, edge_bs, node_Ws, node_bs, glob_Ws, glob_bs):
    src = edge_index[0].astype(jnp.int32)
    dst = edge_index[1].astype(jnp.int32)
    r2 = lambda b: b.reshape(1, -1)

    w1 = edge_Ws[0]
    w1e, w1s, w1d, w1g = w1[:_DE], w1[_DE:_DE + _D], w1[_DE + _D:_DE + 2 * _D], w1[_DE + 2 * _D:]
    wn1 = node_Ws[0]
    wnx, wna, wng = wn1[:_D], wn1[_D:2 * _D], wn1[2 * _D:]
    wg1 = glob_Ws[0]
    wgn, wge, wgg = wg1[:_D], wg1[_D:2 * _D], wg1[2 * _D:]

    ones = jnp.ones((_D, _D), jnp.float32)
    onesb = jnp.ones((_D, _D), jnp.bfloat16)
    ew = (edge_Ws[1].astype(jnp.bfloat16), edge_Ws[2].astype(jnp.bfloat16),
          edge_Ws[3].astype(jnp.bfloat16))
    eb = (r2(edge_bs[1]), r2(edge_bs[2]), r2(edge_bs[3]))
    xs, xd, xn, ge, gn = _prep(x, w1s, w1d, wnx, graph_globals,
                               w1g, wng, r2(edge_bs[0]), r2(node_bs[0]))
    ea_t = edge_attr.T
    parts = ((0, 115200), (115200, 102400), (217600, 102400))
    e_out = None
    esums = []
    aggps = []
    for ebase, size in parts:
        xsum_h = _sc_gather(xs, xd, src, dst, ebase, size)
        e_out, es, e_priv = _edge_mlp(ea_t, xsum_h, w1e.astype(jnp.bfloat16),
                                      *ew, *eb, ge, onesb,
                                      ebase // _BE, eprev=e_out)
        esums.append(es)
        aggps.append(_sc_scatter(e_priv, dst, ebase, size))
    n_out, nsum = _node_mlp(xn, aggps, wna,
                            node_Ws[1], node_Ws[2], node_Ws[3],
                            r2(node_bs[1]), r2(node_bs[2]), r2(node_bs[3]), gn,
                            ones)
    g_out = _glob_mlp(nsum, esums, graph_globals, wgn, wge, wgg,
                      glob_Ws[1], glob_Ws[2], glob_Ws[3],
                      r2(glob_bs[0]), r2(glob_bs[1]), r2(glob_bs[2]), r2(glob_bs[3]))
    return (n_out, e_out, g_out)


# speedup vs baseline: 1.0094x; 1.0094x over previous
import functools

import jax
import jax.numpy as jnp
from jax import lax
from jax.experimental import pallas as pl
from jax.experimental.pallas import tpu as pltpu
from jax.experimental.pallas import tpu_sc as plsc

_N = 10000
_E = 320000
_D = 128
_DE = 16


def _ln_relu(h):
    m = jnp.mean(h, axis=-1, keepdims=True)
    v = jnp.mean((h - m) * (h - m), axis=-1, keepdims=True)
    return jnp.maximum((h - m) / jnp.sqrt(v + 1e-5), 0.0)


def _ln_relu_mxu(h, ones):
    s = jnp.dot(h, ones, preferred_element_type=jnp.float32)
    q = jnp.dot(h * h, ones, preferred_element_type=jnp.float32)
    m = s * (1.0 / _D)
    v = q * (1.0 / _D) - m * m
    return jnp.maximum((h - m) * lax.rsqrt(v + 1e-5), 0.0)


def _dot(a, b):
    return jnp.dot(a, b, preferred_element_type=jnp.float32)


_BN_PREP = 2000


def _prep_body(x_ref, ws_ref, wd_ref, wn_ref, g_ref, wge_ref, wgn_ref,
               be_ref, bn_ref, xs_ref, xd_ref, xn_ref, ge_ref, gn_ref):
    xb = x_ref[...]
    xs_ref[...] = _dot(xb, ws_ref[...])
    xd_ref[...] = _dot(xb, wd_ref[...])
    xn_ref[...] = _dot(xb, wn_ref[...])

    @pl.when(pl.program_id(0) == 0)
    def _():
        g = g_ref[...]
        ge_ref[...] = _dot(g, wge_ref[...]) + be_ref[...]
        gn_ref[...] = _dot(g, wgn_ref[...]) + bn_ref[...]


def _prep(x, w1s, w1d, wnx, g, w1g, wng, be1, bn1):
    full = lambda shp: pl.BlockSpec(shp, lambda i: (0, 0))
    return pl.pallas_call(
        _prep_body,
        grid=(_N // _BN_PREP,),
        in_specs=[
            pl.BlockSpec((_BN_PREP, _D), lambda i: (i, 0)),
            full((_D, _D)), full((_D, _D)), full((_D, _D)),
            full((1, _D)), full((_D, _D)), full((_D, _D)),
            full((1, _D)), full((1, _D)),
        ],
        out_specs=[
            pl.BlockSpec((_BN_PREP, _D), lambda i: (i, 0)),
            pl.BlockSpec((_BN_PREP, _D), lambda i: (i, 0)),
            pl.BlockSpec((_BN_PREP, _D), lambda i: (i, 0)),
            full((1, _D)), full((1, _D)),
        ],
        out_shape=[
            jax.ShapeDtypeStruct((_N, _D), jnp.float32),
            jax.ShapeDtypeStruct((_N, _D), jnp.float32),
            jax.ShapeDtypeStruct((_N, _D), jnp.float32),
            jax.ShapeDtypeStruct((1, _D), jnp.float32),
            jax.ShapeDtypeStruct((1, _D), jnp.float32),
        ],
    )(x, w1s, w1d, wnx, g, w1g, wng, be1, bn1)


_NW = 32
_EPW = _E // _NW
_GC = 80
_GNC = _EPW // _GC


def _sc_gather(xs, xd, src, dst, ebase, size):
    epw = size // _NW
    gnc = epw // _GC
    mesh = plsc.VectorSubcoreMesh(core_axis_name="c", subcore_axis_name="s")

    @functools.partial(
        pl.kernel,
        out_type=jax.ShapeDtypeStruct((size, _D), jnp.float32),
        mesh=mesh,
        scratch_types=[
            pltpu.VMEM((epw,), jnp.int32),
            pltpu.VMEM((epw,), jnp.int32),
            pltpu.VMEM((_GC, _D), jnp.float32),
            pltpu.VMEM((_GC, _D), jnp.float32),
            pltpu.VMEM((_GC, _D), jnp.float32),
            pltpu.VMEM((_GC, _D), jnp.float32),
            pltpu.SemaphoreType.DMA,
            pltpu.SemaphoreType.DMA,
            pltpu.SemaphoreType.DMA,
            pltpu.SemaphoreType.DMA,
            pltpu.SemaphoreType.DMA,
            pltpu.SemaphoreType.DMA,
        ],
    )
    def k(xs_hbm, xd_hbm, src_hbm, dst_hbm, out_hbm,
          ias, iad, rs0, rs1, rd0, rd1, sgs0, sgs1, sgd0, sgd1, so0, so1):
        c = lax.axis_index("c")
        s = lax.axis_index("s")
        base = (s * 2 + c) * epw
        rs, rd = (rs0, rs1), (rd0, rd1)
        sgs, sgd, so = (sgs0, sgs1), (sgd0, sgd1), (so0, so1)

        pltpu.sync_copy(src_hbm.at[pl.ds(ebase + base, epw)], ias)
        pltpu.sync_copy(dst_hbm.at[pl.ds(ebase + base, epw)], iad)

        def issue_gather(j, p):
            pltpu.async_copy(xs_hbm.at[ias.at[pl.ds(j * _GC, _GC)]], rs[p], sgs[p])
            pltpu.async_copy(xd_hbm.at[iad.at[pl.ds(j * _GC, _GC)]], rd[p], sgd[p])

        issue_gather(0, 0)

        def body(m, carry):
            for p in (0, 1):
                j = 2 * m + p
                p1 = 1 - p

                @pl.when(j <= gnc - 1)
                def _():
                    pltpu.make_async_copy(
                        xs_hbm.at[ias.at[pl.ds(0, _GC)]], rs[p], sgs[p]).wait()
                    pltpu.make_async_copy(
                        xd_hbm.at[iad.at[pl.ds(0, _GC)]], rd[p], sgd[p]).wait()

                    @pl.when(j >= 1)
                    def _():
                        pltpu.make_async_copy(
                            rs[p1], out_hbm.at[pl.ds(base, _GC)], so[p1]).wait()

                    @pl.when(j + 1 <= gnc - 1)
                    def _():
                        issue_gather(j + 1, p1)

                    def addrow(i, c2):
                        for kk in range(_D // 16):
                            sl = pl.ds(kk * 16, 16)
                            rs[p][i, sl] = rs[p][i, sl] + rd[p][i, sl]
                        return c2

                    lax.fori_loop(0, _GC, addrow, 0)
                    pltpu.async_copy(
                        rs[p], out_hbm.at[pl.ds(base + j * _GC, _GC)], so[p])
            return carry

        lax.fori_loop(0, (gnc + 1) // 2, body, 0)
        pltpu.make_async_copy(
            rs[(gnc - 1) % 2], out_hbm.at[pl.ds(base, _GC)],
            so[(gnc - 1) % 2]).wait()

    return k(xs, xd, src, dst)


_AGG_NC = _N // _GC


def _sc_scatter(eout, dst, ebase, size):
    epw = size // _NW
    gnc = epw // _GC
    mesh = plsc.VectorSubcoreMesh(core_axis_name="c", subcore_axis_name="s")

    @functools.partial(
        pl.kernel,
        out_type=jax.ShapeDtypeStruct((2, _N, _D), jnp.float32),
        mesh=mesh,
        scratch_types=[
            pltpu.VMEM((_GC,), jnp.int32),
            pltpu.VMEM((_GC,), jnp.int32),
            pltpu.VMEM((_GC, _D), jnp.float32),
            pltpu.VMEM((_GC, _D), jnp.float32),
            pltpu.VMEM_SHARED((_N, _D), jnp.float32),
            pltpu.SemaphoreType.DMA,
            pltpu.SemaphoreType.DMA,
            pltpu.SemaphoreType.DMA,
            pltpu.SemaphoreType.DMA,
        ],
    )
    def k(eout_hbm, dst_hbm, out_hbm, i0, i1, r0, r1, agg_sh, sr0, sr1, si0, si1):
        c = lax.axis_index("c")
        s = lax.axis_index("s")
        ib, rb = (i0, i1), (r0, r1)
        sr, si = (sr0, sr1), (si0, si1)
        nq = jnp.where(s <= (_AGG_NC - 1) % 16, _AGG_NC // 16 + 1, _AGG_NC // 16)

        def zrow(i, carry):
            for kk in range(_D // 16):
                r0[i, pl.ds(kk * 16, 16)] = jnp.zeros((16,), jnp.float32)
            return carry

        lax.fori_loop(0, _GC, zrow, 0)

        def zchunk(m, carry):
            q = s + 16 * m
            pltpu.sync_copy(r0, agg_sh.at[pl.ds(q * _GC, _GC)])
            return carry

        lax.fori_loop(0, nq, zchunk, 0)
        plsc.subcore_barrier()

        w = c * 16 + s
        base = w * epw
        pltpu.async_copy(dst_hbm.at[pl.ds(ebase + base, _GC)], i0, si0)
        pltpu.async_copy(eout_hbm.at[pl.ds(base, _GC)], r0, sr0)

        def body(m, carry):
            for p in (0, 1):
                j = 2 * m + p
                p1 = 1 - p

                @pl.when(j <= gnc - 1)
                def _():
                    pltpu.make_async_copy(
                        dst_hbm.at[pl.ds(base, _GC)], ib[p], si[p]).wait()
                    pltpu.make_async_copy(
                        eout_hbm.at[pl.ds(base, _GC)], rb[p], sr[p]).wait()

                    @pl.when(j + 1 <= gnc - 1)
                    def _():
                        off1 = base + (j + 1) * _GC
                        pltpu.async_copy(dst_hbm.at[pl.ds(ebase + off1, _GC)],
                                         ib[p1], si[p1])
                        pltpu.async_copy(eout_hbm.at[pl.ds(off1, _GC)],
                                         rb[p1], sr[p1])

                    pltpu.sync_copy(rb[p], agg_sh.at[ib[p]], add=True)
            return carry

        lax.fori_loop(0, (gnc + 1) // 2, body, 0)
        plsc.subcore_barrier()

        def wb(m, carry):
            q = s + 16 * m
            pltpu.sync_copy(agg_sh.at[pl.ds(q * _GC, _GC)],
                            out_hbm.at[c, pl.ds(q * _GC, _GC)])
            return carry

        lax.fori_loop(0, nq, wb, 0)

    return k(eout, dst)


_BE = 6400
_S0 = 26 * _BE
_S1 = _E - _S0


def _edge_body(ea_ref, xsum_ref, we_ref, w2_ref, w3_ref, w4_ref,
               b2_ref, b3_ref, b4_ref, ge_ref, ones_ref, eout_ref, esum_ref,
               epriv_ref):
    ones = ones_ref[...]
    ea_part = lax.dot_general(ea_ref[...], we_ref[...],
                              (((0,), (0,)), ((), ())),
                              preferred_element_type=jnp.float32)
    h = ea_part + xsum_ref[...] + ge_ref[...]
    h = _ln_relu_mxu(h, ones)
    h = _ln_relu_mxu(_dot(h, w2_ref[...]) + b2_ref[...], ones)
    h = _ln_relu_mxu(_dot(h, w3_ref[...]) + b3_ref[...], ones)
    out = _dot(h, w4_ref[...]) + b4_ref[...]
    eout_ref[...] = out
    epriv_ref[...] = out

    @pl.when(pl.program_id(0) == 0)
    def _():
        esum_ref[...] = jnp.zeros_like(esum_ref)

    esum_ref[...] += jnp.sum(out, axis=0, keepdims=True)


def _edge_body_alias(eprev_ref, *refs):
    _edge_body(*refs)


def _sum6(a0, a1, a2):
    return (a0[0] + a0[1]) + (a1[0] + a1[1]) + (a2[0] + a2[1])


def _edge_mlp(ea_t, xsum_h, w1e, w2, w3, w4, b2, b3, b4, ge, ones,
              blk_off, eprev=None):
    full = lambda shp: pl.BlockSpec(shp, lambda i: (0, 0))
    nblk = xsum_h.shape[0] // _BE
    in_specs = [
        pl.BlockSpec((_DE, _BE), lambda i, o=blk_off: (0, i + o)),
        pl.BlockSpec((_BE, _D), lambda i: (i, 0)),
        full((_DE, _D)), full((_D, _D)), full((_D, _D)), full((_D, _D)),
        full((1, _D)), full((1, _D)), full((1, _D)), full((1, _D)),
        full((_D, _D)),
    ]
    args = (ea_t, xsum_h, w1e, w2, w3, w4, b2, b3, b4, ge, ones)
    body = _edge_body
    aliases = {}
    if eprev is not None:
        in_specs = [pl.BlockSpec(memory_space=pl.ANY)] + in_specs
        args = (eprev,) + args
        body = _edge_body_alias
        aliases = {0: 0}
    return pl.pallas_call(
        body,
        grid=(nblk,),
        in_specs=in_specs,
        out_specs=[
            pl.BlockSpec((_BE, _D), lambda i, o=blk_off: (i + o, 0)),
            full((1, _D)),
            pl.BlockSpec((_BE, _D), lambda i: (i, 0)),
        ],
        out_shape=[
            jax.ShapeDtypeStruct((_E, _D), jnp.float32),
            jax.ShapeDtypeStruct((1, _D), jnp.float32),
            jax.ShapeDtypeStruct((nblk * _BE, _D), jnp.float32),
        ],
        input_output_aliases=aliases,
    )(*args)


_BN = 2000


def _node_body(xn_ref, aggp0_ref, aggp1_ref, aggp2_ref, wa_ref, w2_ref,
               w3_ref, w4_ref, b2_ref, b3_ref, b4_ref, gn_ref, ones_ref,
               nout_ref, nsum_ref):
    ones = ones_ref[...]
    agg = _sum6(aggp0_ref, aggp1_ref, aggp2_ref)
    h = xn_ref[...] + _dot(agg, wa_ref[...]) + gn_ref[...]
    h = _ln_relu_mxu(h, ones)
    h = _ln_relu_mxu(_dot(h, w2_ref[...]) + b2_ref[...], ones)
    h = _ln_relu_mxu(_dot(h, w3_ref[...]) + b3_ref[...], ones)
    out = _dot(h, w4_ref[...]) + b4_ref[...]
    nout_ref[...] = out

    @pl.when(pl.program_id(0) == 0)
    def _():
        nsum_ref[...] = jnp.zeros_like(nsum_ref)

    nsum_ref[...] += jnp.sum(out, axis=0, keepdims=True)


def _node_mlp(xn, aggps, wna, w2, w3, w4, b2, b3, b4, gn, ones):
    full = lambda shp: pl.BlockSpec(shp, lambda i: (0, 0))
    return pl.pallas_call(
        _node_body,
        grid=(_N // _BN,),
        in_specs=[
            pl.BlockSpec((_BN, _D), lambda i: (i, 0)),
            pl.BlockSpec((2, _BN, _D), lambda i: (0, i, 0)),
            pl.BlockSpec((2, _BN, _D), lambda i: (0, i, 0)),
            pl.BlockSpec((2, _BN, _D), lambda i: (0, i, 0)),
            full((_D, _D)), full((_D, _D)), full((_D, _D)), full((_D, _D)),
            full((1, _D)), full((1, _D)), full((1, _D)), full((1, _D)),
            full((_D, _D)),
        ],
        out_specs=[
            pl.BlockSpec((_BN, _D), lambda i: (i, 0)),
            full((1, _D)),
        ],
        out_shape=[
            jax.ShapeDtypeStruct((_N, _D), jnp.float32),
            jax.ShapeDtypeStruct((1, _D), jnp.float32),
        ],
    )(xn, *aggps, wna, w2, w3, w4, b2, b3, b4, gn, ones)


def _glob_body(nsum_ref, esum0_ref, esum1_ref, esum2_ref, g_ref, wgn_ref,
               wge_ref, wgg_ref, w2_ref, w3_ref, w4_ref, b1_ref, b2_ref,
               b3_ref, b4_ref, gout_ref):
    mn = nsum_ref[...] * (1.0 / _N)
    me = (esum0_ref[...] + esum1_ref[...] + esum2_ref[...]) * (1.0 / _E)
    h = (_dot(mn, wgn_ref[...]) + _dot(me, wge_ref[...])
         + _dot(g_ref[...], wgg_ref[...]) + b1_ref[...])
    h = _ln_relu(h)
    h = _ln_relu(_dot(h, w2_ref[...]) + b2_ref[...])
    h = _ln_relu(_dot(h, w3_ref[...]) + b3_ref[...])
    gout_ref[...] = _dot(h, w4_ref[...]) + b4_ref[...]


def _glob_mlp(nsum, esums, g, wgn, wge, wgg, w2, w3, w4, b1, b2, b3, b4):
    return pl.pallas_call(
        _glob_body,
        out_shape=jax.ShapeDtypeStruct((1, _D), jnp.float32),
    )(nsum, *esums, g, wgn, wge, wgg, w2, w3, w4, b1, b2, b3, b4)


def kernel(x, edge_index, edge_attr, graph_globals,
           edge_Ws, edge_bs, node_Ws, node_bs, glob_Ws, glob_bs):
    src = edge_index[0].astype(jnp.int32)
    dst = edge_index[1].astype(jnp.int32)
    r2 = lambda b: b.reshape(1, -1)

    w1 = edge_Ws[0]
    w1e, w1s, w1d, w1g = w1[:_DE], w1[_DE:_DE + _D], w1[_DE + _D:_DE + 2 * _D], w1[_DE + 2 * _D:]
    wn1 = node_Ws[0]
    wnx, wna, wng = wn1[:_D], wn1[_D:2 * _D], wn1[2 * _D:]
    wg1 = glob_Ws[0]
    wgn, wge, wgg = wg1[:_D], wg1[_D:2 * _D], wg1[2 * _D:]

    ones = jnp.ones((_D, _D), jnp.float32)
    ew = (edge_Ws[1], edge_Ws[2], edge_Ws[3])
    eb = (r2(edge_bs[1]), r2(edge_bs[2]), r2(edge_bs[3]))
    xs, xd, xn, ge, gn = _prep(x, w1s, w1d, wnx, graph_globals,
                               w1g, wng, r2(edge_bs[0]), r2(node_bs[0]))
    ea_t = edge_attr.T
    parts = ((0, 115200), (115200, 102400), (217600, 102400))
    e_out = None
    esums = []
    aggps = []
    for ebase, size in parts:
        xsum_h = _sc_gather(xs, xd, src, dst, ebase, size)
        e_out, es, e_priv = _edge_mlp(ea_t, xsum_h, w1e, *ew, *eb, ge, ones,
                                      ebase // _BE, eprev=e_out)
        esums.append(es)
        aggps.append(_sc_scatter(e_priv, dst, ebase, size))
    n_out, nsum = _node_mlp(xn, aggps, wna,
                            node_Ws[1], node_Ws[2], node_Ws[3],
                            r2(node_bs[1]), r2(node_bs[2]), r2(node_bs[3]), gn,
                            ones)
    g_out = _glob_mlp(nsum, esums, graph_globals, wgn, wge, wgg,
                      glob_Ws[1], glob_Ws[2], glob_Ws[3],
                      r2(glob_bs[0]), r2(glob_bs[1]), r2(glob_bs[2]), r2(glob_bs[3]))
    return (n_out, e_out, g_out)
